# Initial kernel scaffold; baseline (speedup 1.0000x reference)
import jax
import jax.numpy as jnp
from jax.experimental import pallas as pl


def kernel(x):
    raise NotImplementedError("write your pallas kernel here")


# speedup vs baseline: 2.3105x; 2.3105x over previous
import jax
import jax.numpy as jnp
from jax import lax
from jax.experimental import pallas as pl
from jax.experimental.pallas import tpu as pltpu

_ROWS_PER_BLOCK = 16
_N = 32768


def _topk3_body(x_ref, v_ref, i_ref):
    x = x_ref[...]
    iota = lax.broadcasted_iota(jnp.int32, x.shape, 1)
    big = jnp.int32(2**30)
    neg = jnp.float32(-jnp.inf)
    out_iota = lax.broadcasted_iota(jnp.int32, (x.shape[0], 128), 1)
    vvals = jnp.zeros((x.shape[0], 128), jnp.float32)
    ivals = jnp.zeros((x.shape[0], 128), jnp.int32)
    for k in range(3):
        v = jnp.max(x, axis=1, keepdims=True)
        i = jnp.min(jnp.where(x == v, iota, big), axis=1, keepdims=True)
        vvals = jnp.where(out_iota == k, v, vvals)
        ivals = jnp.where(out_iota == k, i, ivals)
        if k < 2:
            x = jnp.where(iota == i, neg, x)
    v_ref[...] = vvals
    i_ref[...] = ivals


def kernel(x):
    m = x.shape[0]
    grid = (m // _ROWS_PER_BLOCK,)
    v, i = pl.pallas_call(
        _topk3_body,
        grid=grid,
        in_specs=[pl.BlockSpec((_ROWS_PER_BLOCK, _N), lambda r: (r, 0))],
        out_specs=[
            pl.BlockSpec((_ROWS_PER_BLOCK, 128), lambda r: (r, 0)),
            pl.BlockSpec((_ROWS_PER_BLOCK, 128), lambda r: (r, 0)),
        ],
        out_shape=[
            jax.ShapeDtypeStruct((m, 128), jnp.float32),
            jax.ShapeDtypeStruct((m, 128), jnp.int32),
        ],
    )(x)
    return (v[:, :3], i[:, :3])
